# baseline (speedup 1.0000x reference)
import jax
import jax.numpy as jnp
from jax.experimental import pallas as pl
from jax.experimental.pallas import tpu as pltpu
from jax.experimental.pallas import tpu_sc as plsc

NP = 5120
IMG = 512.0
SCORE_T = 0.2
IOU_T = 0.2
NEG = -jnp.inf


def _decode_col(cls, anc, loc, n_real):
    s = jnp.max(cls, axis=1, keepdims=True)
    iota_c = jax.lax.broadcasted_iota(jnp.int32, cls.shape, 1)
    c = jnp.min(jnp.where(cls == s, iota_c, 10 ** 9),
                axis=1, keepdims=True).astype(jnp.float32)

    a0, a1, a2, a3 = (anc[:, i:i + 1] for i in range(4))
    r0, r1, r2, r3 = (loc[:, i:i + 1] for i in range(4))
    yca = (a0 + a2) / 2.0
    xca = (a1 + a3) / 2.0
    ha = a2 - a0
    wa = a3 - a1
    w = jnp.exp(r3) * wa
    h = jnp.exp(r2) * ha
    yc = r0 * ha + yca
    xc = r1 * wa + xca
    x1 = jnp.maximum(xc - w / 2.0, 0.0)
    y1 = jnp.maximum(yc - h / 2.0, 0.0)
    x2 = jnp.minimum(xc + w / 2.0, IMG)
    y2 = jnp.minimum(yc + h / 2.0, IMG)

    idx = jax.lax.broadcasted_iota(jnp.int32, (cls.shape[0], 1), 0)
    real = idx < n_real
    valid = jnp.logical_and(s > SCORE_T, real)
    ms = jnp.where(valid, s, NEG)
    coord_max = jnp.max(jnp.where(real, jnp.maximum(jnp.maximum(x1, y1),
                                                    jnp.maximum(x2, y2)), NEG))
    return dict(s=s, c=c, x1=x1, y1=y1, x2=x2, y2=y2, ms=ms, valid=valid,
                coord_max=coord_max)


def _decode_row(cls, anc, loc, n_real):
    s = jnp.max(cls, axis=0, keepdims=True)
    iota_c = jax.lax.broadcasted_iota(jnp.int32, cls.shape, 0)
    c = jnp.min(jnp.where(cls == s, iota_c, 10 ** 9),
                axis=0, keepdims=True).astype(jnp.float32)

    a0, a1, a2, a3 = (anc[i:i + 1, :] for i in range(4))
    r0, r1, r2, r3 = (loc[i:i + 1, :] for i in range(4))
    yca = (a0 + a2) / 2.0
    xca = (a1 + a3) / 2.0
    ha = a2 - a0
    wa = a3 - a1
    w = jnp.exp(r3) * wa
    h = jnp.exp(r2) * ha
    yc = r0 * ha + yca
    xc = r1 * wa + xca
    x1 = jnp.maximum(xc - w / 2.0, 0.0)
    y1 = jnp.maximum(yc - h / 2.0, 0.0)
    x2 = jnp.minimum(xc + w / 2.0, IMG)
    y2 = jnp.minimum(yc + h / 2.0, IMG)

    idx = jax.lax.broadcasted_iota(jnp.int32, (1, cls.shape[1]), 1)
    valid = jnp.logical_and(s > SCORE_T, idx < n_real)
    ms = jnp.where(valid, s, NEG)
    return dict(s=s, c=c, x1=x1, y1=y1, x2=x2, y2=y2, ms=ms, valid=valid)


def _nms_body(cls_c_ref, cls_r_ref, al_c_ref, al_r_ref, rec_ref, dest_ref,
              aw_ref, hw_ref, kcol_ref, acc_ref, rall_ref, kp_ref):
    n_real = 5000

    col = _decode_col(cls_c_ref[0], al_c_ref[0, :, 0:4], al_c_ref[0, :, 4:8],
                      n_real)
    row = _decode_row(cls_r_ref[0], al_r_ref[0, 0:8], al_r_ref[0, 8:16],
                      n_real)

    m1 = col["coord_max"] + 1.0
    o_c = col["c"] * m1
    ox1_c = col["x1"] + o_c
    oy1_c = col["y1"] + o_c
    ox2_c = col["x2"] + o_c
    oy2_c = col["y2"] + o_c
    oar_c = (ox2_c - ox1_c) * (oy2_c - oy1_c)

    o_r = row["c"] * m1
    ox1_r = row["x1"] + o_r
    oy1_r = row["y1"] + o_r
    ox2_r = row["x2"] + o_r
    oy2_r = row["y2"] + o_r
    oar_r = (ox2_r - ox1_r) * (oy2_r - oy1_r)

    rec_ref[0, :, 0:1] = ox1_c
    rec_ref[0, :, 1:2] = oy1_c
    rec_ref[0, :, 2:3] = ox2_c
    rec_ref[0, :, 3:4] = oy2_c
    rec_ref[0, :, 4:5] = oar_c
    rec_ref[0, :, 5:6] = col["ms"]
    rec_ref[0, :, 6:7] = col["x1"]
    rec_ref[0, :, 7:8] = col["y1"]
    rec_ref[0, :, 8:9] = col["x2"]
    rec_ref[0, :, 9:10] = col["y2"]
    rec_ref[0, :, 10:11] = col["s"]
    rec_ref[0, :, 11:12] = col["c"]
    rec_ref[0, :, 12:128] = jnp.zeros((NP, 116), jnp.float32)

    ms_r = row["ms"]
    v_row = row["valid"].astype(jnp.float32)

    iota_r = jax.lax.broadcasted_iota(jnp.int32, (1, NP), 1)
    iota_32 = jax.lax.broadcasted_iota(jnp.int32, (32, 1), 0)
    TI, CH = 32, 512
    NW = NP // 16

    w16 = jnp.left_shift(
        1, jax.lax.broadcasted_iota(jnp.int32, (16, 1), 0)).astype(jnp.float32)

    def gen_tile(mm, _):
        i0 = mm * 128
        for c0 in range(0, NP, CH):
            sl = slice(c0, c0 + CH)
            words = []
            hwords = []
            for q in range(4):
                iq = i0 + TI * q
                ox1i = rec_ref[0, pl.ds(iq, TI), 0:1]
                oy1i = rec_ref[0, pl.ds(iq, TI), 1:2]
                ox2i = rec_ref[0, pl.ds(iq, TI), 2:3]
                oy2i = rec_ref[0, pl.ds(iq, TI), 3:4]
                oari = rec_ref[0, pl.ds(iq, TI), 4:5]
                msi = rec_ref[0, pl.ds(iq, TI), 5:6]
                idxi = iota_32 + iq

                xx1 = jnp.maximum(ox1i, ox1_r[:, sl])
                yy1 = jnp.maximum(oy1i, oy1_r[:, sl])
                xx2 = jnp.minimum(ox2i, ox2_r[:, sl])
                yy2 = jnp.minimum(oy2i, oy2_r[:, sl])
                inter = jnp.maximum(xx2 - xx1, 0.0) * jnp.maximum(yy2 - yy1, 0.0)
                union = oari + oar_r[:, sl] - inter
                over = jnp.logical_and(inter > IOU_T * union, union > 0.0)
                hi = jnp.logical_or(
                    msi > ms_r[:, sl],
                    jnp.logical_and(msi == ms_r[:, sl], idxi < iota_r[:, sl]))
                a = jnp.logical_and(over, hi).astype(jnp.float32)
                hif = hi.astype(jnp.float32)
                words.append(jnp.sum(a[0:16, :] * w16, axis=0, keepdims=True))
                words.append(jnp.sum(a[16:32, :] * w16, axis=0, keepdims=True))
                hwords.append(jnp.sum(hif[0:16, :] * w16, axis=0,
                                      keepdims=True))
                hwords.append(jnp.sum(hif[16:32, :] * w16, axis=0,
                                      keepdims=True))
                rall_ref[:, sl] = rall_ref[:, sl] + jnp.sum(
                    hif, axis=0, keepdims=True)
            pk = jnp.concatenate(words, axis=0)
            aw_ref[pl.ds(mm * 8, 8), sl] = pk.astype(jnp.int32)
            hw_ref[pl.ds(mm * 8, 8), sl] = jnp.concatenate(
                hwords, axis=0).astype(jnp.int32)
        return 0

    rall_ref[...] = jnp.zeros((1, NP), jnp.float32)
    jax.lax.fori_loop(0, NP // 128, gen_tile, 0)

    def store_col(k):
        kt = jnp.transpose(jnp.reshape(k, (NP // 128, 128)), (1, 0))
        for s in range(NP // 128):
            kcol_ref[s * 128:(s + 1) * 128, 0:1] = kt[:, s:s + 1]

    wlane = jnp.left_shift(
        1, jax.lax.broadcasted_iota(jnp.int32, (1, 128), 1) % 16
    ).astype(jnp.float32)

    def store_kp(k):
        m = jnp.reshape(k, (NP // 128, 128)) * wlane
        kp40x8 = jnp.concatenate(
            [jnp.sum(m[:, g * 16:(g + 1) * 16], axis=1, keepdims=True)
             for g in range(8)], axis=1)
        t8x40 = jnp.transpose(kp40x8, (1, 0)).astype(jnp.int32)
        for s in range(NP // 128):
            kp_ref[s * 8:(s + 1) * 8, 0:1] = t8x40[:, s:s + 1]

    store_kp(v_row)

    def fp_cond(carry):
        k_row, t, changed = carry
        return jnp.logical_and(changed, t < NP)

    def fp_body(carry):
        k_row, t, _ = carry
        changed = jnp.bool_(False)
        chunks = []
        for c0 in range(0, NP, CH):
            def blk(mm, sup, c0=c0):
                w0 = mm * TI
                kc = kp_ref[pl.ds(w0, TI), 0:1]
                tile = aw_ref[pl.ds(w0, TI), c0:c0 + CH]
                hitv = (jnp.bitwise_and(tile, kc) != 0).astype(jnp.float32)
                return jnp.maximum(sup, jnp.max(hitv, axis=0, keepdims=True))

            supp = jax.lax.fori_loop(0, NW // TI, blk,
                                     jnp.zeros((1, CH), jnp.float32))
            k_new_c = jnp.where(supp > 0.0, 0.0, v_row[:, c0:c0 + CH])
            changed = jnp.logical_or(
                changed, jnp.any(k_new_c != k_row[:, c0:c0 + CH]))
            m = jnp.reshape(k_new_c, (CH // 128, 128)) * wlane
            kpc = jnp.concatenate(
                [jnp.sum(m[:, g * 16:(g + 1) * 16], axis=1, keepdims=True)
                 for g in range(8)], axis=1)
            tr = jnp.transpose(kpc, (1, 0)).astype(jnp.int32)
            for s in range(CH // 128):
                r0 = 8 * (c0 // 128 + s)
                kp_ref[r0:r0 + 8, 0:1] = tr[:, s:s + 1]
            chunks.append(k_new_c)
        k_new = jnp.concatenate(chunks, axis=1)
        return k_new, t + 1, changed

    k_row, _, _ = jax.lax.while_loop(
        fp_cond, fp_body, (v_row, jnp.int32(0), jnp.bool_(True)))
    store_col(k_row)

    acc_ref[...] = jnp.zeros((1, NP), jnp.float32)

    def rkept_tile(mm, _):
        w0 = mm * TI
        kc = kp_ref[pl.ds(w0, TI), 0:1]
        for c0 in range(0, NP, CH):
            t = jnp.bitwise_and(hw_ref[pl.ds(w0, TI), c0:c0 + CH], kc)
            pc = jax.lax.population_count(t).astype(jnp.float32)
            acc_ref[:, c0:c0 + CH] = acc_ref[:, c0:c0 + CH] + jnp.sum(
                pc, axis=0, keepdims=True)
        return 0

    jax.lax.fori_loop(0, NW // TI, rkept_tile, 0)
    r_kept = acc_ref[...]
    cnt = jnp.sum(k_row)
    dest = jnp.where(k_row > 0.0, r_kept, cnt + (rall_ref[...] - r_kept))
    dest_ref[...] = jnp.reshape(dest, (1, 1, NP))

    fill_lane = jax.lax.broadcasted_iota(jnp.int32, (1, 128), 1)
    filler = jnp.where(fill_lane == 5, -1.0, 0.0)

    def payload(cc, _):
        r0 = cc * 128
        kc = kcol_ref[pl.ds(r0, 128), 0:1]
        vals = jnp.concatenate(
            [rec_ref[0, pl.ds(r0, 128), 6:12],
             jnp.zeros((128, 122), jnp.float32)], axis=1)
        rec_ref[0, pl.ds(r0, 128), :] = jnp.where(kc > 0.0, vals, filler)
        return 0

    jax.lax.fori_loop(0, NP // 128, payload, 0)


def _nms_kwargs(B):
    return dict(
        grid=(B,),
        in_specs=[
            pl.BlockSpec((1, NP, 128), lambda b: (b, 0, 0)),
            pl.BlockSpec((1, 96, NP), lambda b: (b, 0, 0)),
            pl.BlockSpec((1, NP, 8), lambda b: (b, 0, 0)),
            pl.BlockSpec((1, 16, NP), lambda b: (b, 0, 0)),
        ],
        out_specs=[
            pl.BlockSpec((1, NP, 128), lambda b: (b, 0, 0)),
            pl.BlockSpec((1, 1, NP), lambda b: (b, 0, 0)),
        ],
        out_shape=[
            jax.ShapeDtypeStruct((B, NP, 128), jnp.float32),
            jax.ShapeDtypeStruct((B, 1, NP), jnp.float32),
        ],
        scratch_shapes=[
            pltpu.VMEM((NP // 16, NP), jnp.int32),
            pltpu.VMEM((NP // 16, NP), jnp.int32),
            pltpu.VMEM((NP, 1), jnp.float32),
            pltpu.VMEM((1, NP), jnp.float32),
            pltpu.VMEM((1, NP), jnp.float32),
            pltpu.VMEM((NP // 16, 1), jnp.int32),
        ],
        compiler_params=pltpu.CompilerParams(
            dimension_semantics=("parallel",),
            vmem_limit_bytes=63 * 1024 * 1024),
    )


def _run_nms(cls_c, cls_r, al_c, al_r):
    B = cls_c.shape[0]
    return pl.pallas_call(_nms_body, **_nms_kwargs(B))(cls_c, cls_r, al_c, al_r)


def _sc_scatter(rec_flat, g_dest):
    R = rec_flat.shape[0]
    W = 128
    mesh = plsc.VectorSubcoreMesh(core_axis_name="core",
                                  subcore_axis_name="subcore")

    @pl.kernel(out_type=jax.ShapeDtypeStruct((R, 128), jnp.float32),
               mesh=mesh, scratch_types=[])
    def scatter_kernel(x_hbm, i_hbm, o_hbm):
        def body(x_vmem, i_vmem):
            pltpu.sync_copy(x_vmem, o_hbm.at[i_vmem.at[0]])

        pltpu.emit_pipeline(
            body,
            grid=(R // W,),
            in_specs=[
                pl.BlockSpec((W, 128), index_map=lambda i: (i, 0)),
                pl.BlockSpec((1, W), index_map=lambda i: (0, i)),
            ],
            out_specs=[],
            core_axis_name="subcore",
            dimension_semantics=(pltpu.PARALLEL,),
        )(x_hbm, i_hbm)

    return scatter_kernel(rec_flat, g_dest)


def _prep_inputs(cls_preds, loc_preds, anchors):
    B, N, C = cls_preds.shape
    cls_c = jnp.pad(cls_preds, ((0, 0), (0, NP - N), (0, 128 - C)),
                    constant_values=-1.0)
    cls_r = jnp.pad(jnp.transpose(cls_preds, (0, 2, 1)),
                    ((0, 0), (0, 96 - C), (0, NP - N)), constant_values=-1.0)
    anc_b = jnp.broadcast_to(anchors, (B, N, 4))
    al_c = jnp.pad(jnp.concatenate([anc_b, loc_preds], axis=2),
                   ((0, 0), (0, NP - N), (0, 0)))
    al_r = jnp.pad(jnp.concatenate(
        [jnp.pad(jnp.transpose(anc_b, (0, 2, 1)), ((0, 0), (0, 4), (0, 0))),
         jnp.pad(jnp.transpose(loc_preds, (0, 2, 1)), ((0, 0), (0, 4), (0, 0)))],
        axis=1), ((0, 0), (0, 0), (0, NP - N)))
    return cls_c, cls_r, al_c, al_r


def kernel(cls_preds, loc_preds, anchors):
    B, N, _ = cls_preds.shape
    ins = _prep_inputs(cls_preds, loc_preds, anchors)
    outs = []
    for b in range(B):
        rec_b, dest_b = _run_nms(*(x[b:b + 1] for x in ins))
        g_dest = dest_b[:, 0, :].astype(jnp.int32)
        outs.append(_sc_scatter(rec_b.reshape(NP, 128), g_dest))
    out = jnp.stack(outs)[:, :N]
    boxes = out[..., 0:4]
    scores = out[..., 4]
    labels = out[..., 5].astype(jnp.int32)
    return boxes, scores, labels


# speedup vs baseline: 1.0050x; 1.0050x over previous
import jax
import jax.numpy as jnp
from jax.experimental import pallas as pl
from jax.experimental.pallas import tpu as pltpu
from jax.experimental.pallas import tpu_sc as plsc

NP = 5120
IMG = 512.0
SCORE_T = 0.2
IOU_T = 0.2
NEG = -jnp.inf


def _decode_col(cls, anc, loc, n_real):
    s = jnp.max(cls, axis=1, keepdims=True)
    iota_c = jax.lax.broadcasted_iota(jnp.int32, cls.shape, 1)
    c = jnp.min(jnp.where(cls == s, iota_c, 10 ** 9),
                axis=1, keepdims=True).astype(jnp.float32)

    a0, a1, a2, a3 = (anc[:, i:i + 1] for i in range(4))
    r0, r1, r2, r3 = (loc[:, i:i + 1] for i in range(4))
    yca = (a0 + a2) / 2.0
    xca = (a1 + a3) / 2.0
    ha = a2 - a0
    wa = a3 - a1
    w = jnp.exp(r3) * wa
    h = jnp.exp(r2) * ha
    yc = r0 * ha + yca
    xc = r1 * wa + xca
    x1 = jnp.maximum(xc - w / 2.0, 0.0)
    y1 = jnp.maximum(yc - h / 2.0, 0.0)
    x2 = jnp.minimum(xc + w / 2.0, IMG)
    y2 = jnp.minimum(yc + h / 2.0, IMG)

    idx = jax.lax.broadcasted_iota(jnp.int32, (cls.shape[0], 1), 0)
    real = idx < n_real
    valid = jnp.logical_and(s > SCORE_T, real)
    ms = jnp.where(valid, s, NEG)
    coord_max = jnp.max(jnp.where(real, jnp.maximum(jnp.maximum(x1, y1),
                                                    jnp.maximum(x2, y2)), NEG))
    return dict(s=s, c=c, x1=x1, y1=y1, x2=x2, y2=y2, ms=ms, valid=valid,
                coord_max=coord_max)


def _decode_row(cls, anc, loc, n_real):
    s = jnp.max(cls, axis=0, keepdims=True)
    iota_c = jax.lax.broadcasted_iota(jnp.int32, cls.shape, 0)
    c = jnp.min(jnp.where(cls == s, iota_c, 10 ** 9),
                axis=0, keepdims=True).astype(jnp.float32)

    a0, a1, a2, a3 = (anc[i:i + 1, :] for i in range(4))
    r0, r1, r2, r3 = (loc[i:i + 1, :] for i in range(4))
    yca = (a0 + a2) / 2.0
    xca = (a1 + a3) / 2.0
    ha = a2 - a0
    wa = a3 - a1
    w = jnp.exp(r3) * wa
    h = jnp.exp(r2) * ha
    yc = r0 * ha + yca
    xc = r1 * wa + xca
    x1 = jnp.maximum(xc - w / 2.0, 0.0)
    y1 = jnp.maximum(yc - h / 2.0, 0.0)
    x2 = jnp.minimum(xc + w / 2.0, IMG)
    y2 = jnp.minimum(yc + h / 2.0, IMG)

    idx = jax.lax.broadcasted_iota(jnp.int32, (1, cls.shape[1]), 1)
    valid = jnp.logical_and(s > SCORE_T, idx < n_real)
    ms = jnp.where(valid, s, NEG)
    return dict(s=s, c=c, x1=x1, y1=y1, x2=x2, y2=y2, ms=ms, valid=valid)


def _nms_body(cls_c_ref, cls_r_ref, al_c_ref, al_r_ref, rec_ref, dest_ref,
              aw_ref, hw_ref, kcol_ref, acc_ref, rall_ref, kp_ref):
    n_real = 5000

    col = _decode_col(cls_c_ref[0], al_c_ref[0, :, 0:4], al_c_ref[0, :, 4:8],
                      n_real)
    row = _decode_row(cls_r_ref[0], al_r_ref[0, 0:8], al_r_ref[0, 8:16],
                      n_real)

    m1 = col["coord_max"] + 1.0
    o_c = col["c"] * m1
    ox1_c = col["x1"] + o_c
    oy1_c = col["y1"] + o_c
    ox2_c = col["x2"] + o_c
    oy2_c = col["y2"] + o_c
    oar_c = (ox2_c - ox1_c) * (oy2_c - oy1_c)

    o_r = row["c"] * m1
    ox1_r = row["x1"] + o_r
    oy1_r = row["y1"] + o_r
    ox2_r = row["x2"] + o_r
    oy2_r = row["y2"] + o_r
    oar_r = (ox2_r - ox1_r) * (oy2_r - oy1_r)

    rec_ref[0, :, 0:1] = ox1_c
    rec_ref[0, :, 1:2] = oy1_c
    rec_ref[0, :, 2:3] = ox2_c
    rec_ref[0, :, 3:4] = oy2_c
    rec_ref[0, :, 4:5] = oar_c
    rec_ref[0, :, 5:6] = col["ms"]
    rec_ref[0, :, 6:7] = col["x1"]
    rec_ref[0, :, 7:8] = col["y1"]
    rec_ref[0, :, 8:9] = col["x2"]
    rec_ref[0, :, 9:10] = col["y2"]
    rec_ref[0, :, 10:11] = col["s"]
    rec_ref[0, :, 11:12] = col["c"]
    rec_ref[0, :, 12:128] = jnp.zeros((NP, 116), jnp.float32)

    ms_r = row["ms"]
    v_row = row["valid"].astype(jnp.float32)

    iota_r = jax.lax.broadcasted_iota(jnp.int32, (1, NP), 1)
    iota_32 = jax.lax.broadcasted_iota(jnp.int32, (32, 1), 0)
    TI, CH = 32, 512
    NW = NP // 16

    w16 = jnp.left_shift(
        1, jax.lax.broadcasted_iota(jnp.int32, (16, 1), 0)).astype(jnp.float32)

    def gen_tile(mm, _):
        i0 = mm * 128
        for c0 in range(0, NP, CH):
            sl = slice(c0, c0 + CH)
            words = []
            hwords = []
            for q in range(4):
                iq = i0 + TI * q
                ox1i = rec_ref[0, pl.ds(iq, TI), 0:1]
                oy1i = rec_ref[0, pl.ds(iq, TI), 1:2]
                ox2i = rec_ref[0, pl.ds(iq, TI), 2:3]
                oy2i = rec_ref[0, pl.ds(iq, TI), 3:4]
                oari = rec_ref[0, pl.ds(iq, TI), 4:5]
                msi = rec_ref[0, pl.ds(iq, TI), 5:6]
                idxi = iota_32 + iq

                xx1 = jnp.maximum(ox1i, ox1_r[:, sl])
                yy1 = jnp.maximum(oy1i, oy1_r[:, sl])
                xx2 = jnp.minimum(ox2i, ox2_r[:, sl])
                yy2 = jnp.minimum(oy2i, oy2_r[:, sl])
                inter = jnp.maximum(xx2 - xx1, 0.0) * jnp.maximum(yy2 - yy1, 0.0)
                union = oari + oar_r[:, sl] - inter
                over = jnp.logical_and(inter > IOU_T * union, union > 0.0)
                hi = jnp.logical_or(
                    msi > ms_r[:, sl],
                    jnp.logical_and(msi == ms_r[:, sl], idxi < iota_r[:, sl]))
                a = jnp.logical_and(over, hi).astype(jnp.float32)
                hif = hi.astype(jnp.float32)
                words.append(jnp.sum(a[0:16, :] * w16, axis=0, keepdims=True))
                words.append(jnp.sum(a[16:32, :] * w16, axis=0, keepdims=True))
                hwords.append(jnp.sum(hif[0:16, :] * w16, axis=0,
                                      keepdims=True))
                hwords.append(jnp.sum(hif[16:32, :] * w16, axis=0,
                                      keepdims=True))
                rall_ref[:, sl] = rall_ref[:, sl] + jnp.sum(
                    hif, axis=0, keepdims=True)
            pk = jnp.concatenate(words, axis=0)
            aw_ref[pl.ds(mm * 8, 8), sl] = pk.astype(jnp.int32)
            hw_ref[pl.ds(mm * 8, 8), sl] = jnp.concatenate(
                hwords, axis=0).astype(jnp.int32)
        return 0

    rall_ref[...] = jnp.zeros((1, NP), jnp.float32)
    jax.lax.fori_loop(0, NP // 128, gen_tile, 0)

    def store_col(k):
        kt = jnp.transpose(jnp.reshape(k, (NP // 128, 128)), (1, 0))
        for s in range(NP // 128):
            kcol_ref[s * 128:(s + 1) * 128, 0:1] = kt[:, s:s + 1]

    wlane = jnp.left_shift(
        1, jax.lax.broadcasted_iota(jnp.int32, (1, 128), 1) % 16
    ).astype(jnp.float32)

    def store_kp(k):
        m = jnp.reshape(k, (NP // 128, 128)) * wlane
        kp40x8 = jnp.concatenate(
            [jnp.sum(m[:, g * 16:(g + 1) * 16], axis=1, keepdims=True)
             for g in range(8)], axis=1)
        t8x40 = jnp.transpose(kp40x8, (1, 0)).astype(jnp.int32)
        for s in range(NP // 128):
            kp_ref[s * 8:(s + 1) * 8, 0:1] = t8x40[:, s:s + 1]

    store_kp(v_row)

    def fp_cond(carry):
        k_row, t, changed = carry
        return jnp.logical_and(changed, t < NP)

    def fp_body(carry):
        k_row, t, _ = carry
        changed = jnp.bool_(False)
        chunks = []
        for c0 in range(0, NP, CH):
            def blk(mm, sup, c0=c0):
                w0 = mm * TI
                kc = kp_ref[pl.ds(w0, TI), 0:1]
                tile = aw_ref[pl.ds(w0, TI), c0:c0 + CH]
                hitv = (jnp.bitwise_and(tile, kc) != 0).astype(jnp.float32)
                return jnp.maximum(sup, jnp.max(hitv, axis=0, keepdims=True))

            supp = jax.lax.fori_loop(0, NW // TI, blk,
                                     jnp.zeros((1, CH), jnp.float32))
            k_new_c = jnp.where(supp > 0.0, 0.0, v_row[:, c0:c0 + CH])
            changed = jnp.logical_or(
                changed, jnp.any(k_new_c != k_row[:, c0:c0 + CH]))
            m = jnp.reshape(k_new_c, (CH // 128, 128)) * wlane
            kpc = jnp.concatenate(
                [jnp.sum(m[:, g * 16:(g + 1) * 16], axis=1, keepdims=True)
                 for g in range(8)], axis=1)
            tr = jnp.transpose(kpc, (1, 0)).astype(jnp.int32)
            for s in range(CH // 128):
                r0 = 8 * (c0 // 128 + s)
                kp_ref[r0:r0 + 8, 0:1] = tr[:, s:s + 1]
            chunks.append(k_new_c)
        k_new = jnp.concatenate(chunks, axis=1)
        return k_new, t + 1, changed

    k_row, _, _ = jax.lax.while_loop(
        fp_cond, fp_body, (v_row, jnp.int32(0), jnp.bool_(True)))
    store_col(k_row)

    acc_ref[...] = jnp.zeros((1, NP), jnp.float32)

    def rkept_tile(mm, _):
        w0 = mm * TI
        kc = kp_ref[pl.ds(w0, TI), 0:1]
        for c0 in range(0, NP, CH):
            t = jnp.bitwise_and(hw_ref[pl.ds(w0, TI), c0:c0 + CH], kc)
            pc = jax.lax.population_count(t).astype(jnp.float32)
            acc_ref[:, c0:c0 + CH] = acc_ref[:, c0:c0 + CH] + jnp.sum(
                pc, axis=0, keepdims=True)
        return 0

    jax.lax.fori_loop(0, NW // TI, rkept_tile, 0)
    r_kept = acc_ref[...]
    cnt = jnp.sum(k_row)
    dest = jnp.where(k_row > 0.0, r_kept, cnt + (rall_ref[...] - r_kept))
    dest_ref[...] = jnp.reshape(dest, (1, 1, NP))

    fill_lane = jax.lax.broadcasted_iota(jnp.int32, (1, 128), 1)
    filler = jnp.where(fill_lane == 5, -1.0, 0.0)

    def payload(cc, _):
        r0 = cc * 128
        kc = kcol_ref[pl.ds(r0, 128), 0:1]
        vals = jnp.concatenate(
            [rec_ref[0, pl.ds(r0, 128), 6:12],
             jnp.zeros((128, 122), jnp.float32)], axis=1)
        rec_ref[0, pl.ds(r0, 128), :] = jnp.where(kc > 0.0, vals, filler)
        return 0

    jax.lax.fori_loop(0, NP // 128, payload, 0)


def _nms_kwargs(B):
    return dict(
        grid=(B,),
        in_specs=[
            pl.BlockSpec((1, NP, 128), lambda b: (b, 0, 0)),
            pl.BlockSpec((1, 96, NP), lambda b: (b, 0, 0)),
            pl.BlockSpec((1, NP, 8), lambda b: (b, 0, 0)),
            pl.BlockSpec((1, 16, NP), lambda b: (b, 0, 0)),
        ],
        out_specs=[
            pl.BlockSpec((1, NP, 128), lambda b: (b, 0, 0)),
            pl.BlockSpec((1, 1, NP), lambda b: (b, 0, 0)),
        ],
        out_shape=[
            jax.ShapeDtypeStruct((B, NP, 128), jnp.float32),
            jax.ShapeDtypeStruct((B, 1, NP), jnp.float32),
        ],
        scratch_shapes=[
            pltpu.VMEM((NP // 16, NP), jnp.int32),
            pltpu.VMEM((NP // 16, NP), jnp.int32),
            pltpu.VMEM((NP, 1), jnp.float32),
            pltpu.VMEM((1, NP), jnp.float32),
            pltpu.VMEM((1, NP), jnp.float32),
            pltpu.VMEM((NP // 16, 1), jnp.int32),
        ],
        compiler_params=pltpu.CompilerParams(
            dimension_semantics=("parallel",),
            vmem_limit_bytes=63 * 1024 * 1024),
    )


def _run_nms(cls_c, cls_r, al_c, al_r):
    B = cls_c.shape[0]
    return pl.pallas_call(_nms_body, **_nms_kwargs(B))(cls_c, cls_r, al_c, al_r)


def _sc_scatter(rec_flat, g_dest):
    R = rec_flat.shape[0]
    W = 128
    mesh = plsc.VectorSubcoreMesh(core_axis_name="core",
                                  subcore_axis_name="subcore")

    @pl.kernel(out_type=jax.ShapeDtypeStruct((R, 128), jnp.float32),
               mesh=mesh, scratch_types=[])
    def scatter_kernel(x_hbm, i_hbm, o_hbm):
        def body(x_vmem, i_vmem):
            pltpu.sync_copy(x_vmem, o_hbm.at[i_vmem.at[0]])

        pltpu.emit_pipeline(
            body,
            grid=(R // W,),
            in_specs=[
                pl.BlockSpec((W, 128), index_map=lambda i: (i, 0)),
                pl.BlockSpec((1, W), index_map=lambda i: (0, i)),
            ],
            out_specs=[],
            core_axis_name="subcore",
            dimension_semantics=(pltpu.PARALLEL,),
        )(x_hbm, i_hbm)

    return scatter_kernel(rec_flat, g_dest)


def _prep_inputs(cls_preds, loc_preds, anchors):
    B, N, C = cls_preds.shape
    cls_c = jnp.pad(cls_preds, ((0, 0), (0, NP - N), (0, 128 - C)),
                    constant_values=-1.0)
    cls_r = jnp.pad(jnp.transpose(cls_preds, (0, 2, 1)),
                    ((0, 0), (0, 96 - C), (0, NP - N)), constant_values=-1.0)
    anc_b = jnp.broadcast_to(anchors, (B, N, 4))
    al_c = jnp.pad(jnp.concatenate([anc_b, loc_preds], axis=2),
                   ((0, 0), (0, NP - N), (0, 0)))
    al_r = jnp.pad(jnp.concatenate(
        [jnp.pad(jnp.transpose(anc_b, (0, 2, 1)), ((0, 0), (0, 4), (0, 0))),
         jnp.pad(jnp.transpose(loc_preds, (0, 2, 1)), ((0, 0), (0, 4), (0, 0)))],
        axis=1), ((0, 0), (0, 0), (0, NP - N)))
    return cls_c, cls_r, al_c, al_r


def kernel(cls_preds, loc_preds, anchors):
    B, N, _ = cls_preds.shape
    rec, dest = _run_nms(*_prep_inputs(cls_preds, loc_preds, anchors))
    g_dest = (dest[:, 0, :].astype(jnp.int32)
              + jnp.arange(B, dtype=jnp.int32)[:, None] * NP).reshape(1, B * NP)
    out = _sc_scatter(rec.reshape(B * NP, 128), g_dest)
    out = out.reshape(B, NP, 128)[:, :N]
    boxes = out[..., 0:4]
    scores = out[..., 4]
    labels = out[..., 5].astype(jnp.int32)
    return boxes, scores, labels
